# baseline (speedup 1.0000x reference)
import functools

import jax
import jax.numpy as jnp
from jax.experimental import pallas as pl

_BM = 512


def _proj_body(x_ref, w_ref, asrc_ref, adst_ref,
               whe_ref, g_ref, eb_ref, ed_ref, mean_ref, *, hid):
    n = x_ref.shape[0]
    h = pl.program_id(0)
    wh = jnp.dot(x_ref[...], w_ref[0], preferred_element_type=jnp.float32)
    whe_ref[0, :, :hid] = wh.astype(jnp.bfloat16)
    col = jax.lax.broadcasted_iota(jnp.int32, (n, hid), 1)
    whe_ref[0, :, hid:] = jnp.where(col == 0, 1.0, 0.0).astype(jnp.bfloat16)
    asrc = asrc_ref[h][:, None]
    adst = adst_ref[h][:, None]
    aboth = jnp.concatenate([asrc, adst], axis=1)
    fs = jnp.dot(wh, aboth, preferred_element_type=jnp.float32)
    f_src = fs[:, 0:1]
    f_dst = fs[:, 1:2]
    g_ref[0] = jnp.exp(-0.8 * f_src)
    eb_ref[0] = jnp.exp(f_dst)
    ed_ref[0] = jnp.exp(0.2 * f_dst)
    mean_ref[0] = jnp.mean(wh, axis=0, keepdims=True)


def _attn_body(adj_ref, whe_ref, g_ref, eb_ref, ed_ref, mean_ref,
               out_ref, *, bm, heads, hid):
    i = pl.program_id(0)
    adj = adj_ref[...]
    for h in range(heads):
        g = g_ref[h, pl.ds(i * bm, bm), :]
        eb = eb_ref[h]
        ed = ed_ref[h]
        q = jnp.maximum(eb, g * ed)
        p = adj * q
        pv = jnp.dot(p, whe_ref[h], preferred_element_type=jnp.float32)
        denom = pv[:, hid : hid + 1]
        o = jnp.where(denom > 0.0, pv[:, :hid] / denom, mean_ref[h])
        o = jnp.where(o > 0.0, o, jnp.exp(o) - 1.0)
        out_ref[:, h * hid : (h + 1) * hid] = o


def _gat_layer(x, adj_bf, W, a_src, a_dst):
    n = x.shape[0]
    fin = x.shape[1]
    heads, _, hid = W.shape
    bm = min(_BM, n)
    x = x.astype(jnp.bfloat16)
    W = W.astype(jnp.bfloat16)

    whe, g, eb, ed, wh_mean = pl.pallas_call(
        functools.partial(_proj_body, hid=hid),
        grid=(heads,),
        in_specs=[
            pl.BlockSpec((n, fin), lambda h: (0, 0)),
            pl.BlockSpec((1, fin, hid), lambda h: (h, 0, 0)),
            pl.BlockSpec((heads, hid), lambda h: (0, 0)),
            pl.BlockSpec((heads, hid), lambda h: (0, 0)),
        ],
        out_specs=[
            pl.BlockSpec((1, n, 2 * hid), lambda h: (h, 0, 0)),
            pl.BlockSpec((1, n, 1), lambda h: (h, 0, 0)),
            pl.BlockSpec((1, n, 1), lambda h: (h, 0, 0)),
            pl.BlockSpec((1, n, 1), lambda h: (h, 0, 0)),
            pl.BlockSpec((1, 1, hid), lambda h: (h, 0, 0)),
        ],
        out_shape=[
            jax.ShapeDtypeStruct((heads, n, 2 * hid), jnp.bfloat16),
            jax.ShapeDtypeStruct((heads, n, 1), jnp.float32),
            jax.ShapeDtypeStruct((heads, n, 1), jnp.float32),
            jax.ShapeDtypeStruct((heads, n, 1), jnp.float32),
            jax.ShapeDtypeStruct((heads, 1, hid), jnp.float32),
        ],
    )(x, W, a_src, a_dst)
    g = g.astype(jnp.bfloat16)
    eb = eb.reshape(heads, 1, n).astype(jnp.bfloat16)
    ed = ed.reshape(heads, 1, n).astype(jnp.bfloat16)

    out = pl.pallas_call(
        functools.partial(_attn_body, bm=bm, heads=heads, hid=hid),
        grid=(n // bm,),
        in_specs=[
            pl.BlockSpec((bm, n), lambda i: (i, 0)),
            pl.BlockSpec((heads, n, 2 * hid), lambda i: (0, 0, 0)),
            pl.BlockSpec((heads, n, 1), lambda i: (0, 0, 0)),
            pl.BlockSpec((heads, 1, n), lambda i: (0, 0, 0)),
            pl.BlockSpec((heads, 1, n), lambda i: (0, 0, 0)),
            pl.BlockSpec((heads, 1, hid), lambda i: (0, 0, 0)),
        ],
        out_specs=pl.BlockSpec((bm, heads * hid), lambda i: (i, 0)),
        out_shape=jax.ShapeDtypeStruct((n, heads * hid), jnp.float32),
    )(adj_bf, whe, g, eb, ed, wh_mean)
    return out


@jax.jit
def kernel(features, adj, W1, a_src1, a_dst1, W2, a_src2, a_dst2, W3, a_src3, a_dst3):
    adj_bf = adj.astype(jnp.bfloat16)
    x = _gat_layer(features, adj_bf, W1, a_src1, a_dst1)
    x = _gat_layer(x, adj_bf, W2, a_src2, a_dst2)
    x = _gat_layer(x, adj_bf, W3, a_src3, a_dst3)
    return x


# speedup vs baseline: 1.0210x; 1.0210x over previous
import functools

import jax
import jax.numpy as jnp
from jax.experimental import pallas as pl

_BM = 1024


def _proj_body(x_ref, w_ref, asrc_ref, adst_ref,
               whe_ref, g_ref, eb_ref, ed_ref, mean_ref, *, hid):
    n = x_ref.shape[0]
    h = pl.program_id(0)
    wh = jnp.dot(x_ref[...], w_ref[0], preferred_element_type=jnp.float32)
    whe_ref[0, :, :hid] = wh.astype(jnp.bfloat16)
    col = jax.lax.broadcasted_iota(jnp.int32, (n, hid), 1)
    whe_ref[0, :, hid:] = jnp.where(col == 0, 1.0, 0.0).astype(jnp.bfloat16)
    asrc = asrc_ref[h][:, None]
    adst = adst_ref[h][:, None]
    aboth = jnp.concatenate([asrc, adst], axis=1)
    fs = jnp.dot(wh, aboth, preferred_element_type=jnp.float32)
    f_src = fs[:, 0:1]
    f_dst = fs[:, 1:2]
    g_ref[0] = jnp.exp(-0.8 * f_src)
    eb_ref[0] = jnp.exp(f_dst)
    ed_ref[0] = jnp.exp(0.2 * f_dst)
    mean_ref[0] = jnp.mean(wh, axis=0, keepdims=True)


def _attn_body(adj_ref, whe_ref, g_ref, eb_ref, ed_ref, mean_ref,
               out_ref, *, bm, heads, hid):
    i = pl.program_id(0)
    adj = adj_ref[...]
    for h in range(heads):
        g = g_ref[h, pl.ds(i * bm, bm), :]
        eb = eb_ref[h]
        ed = ed_ref[h]
        q = jnp.maximum(eb, g * ed)
        p = adj * q
        pv = jnp.dot(p, whe_ref[h], preferred_element_type=jnp.float32)
        denom = pv[:, hid : hid + 1]
        o = jnp.where(denom > 0.0, pv[:, :hid] / denom, mean_ref[h])
        o = jnp.where(o > 0.0, o, jnp.exp(o) - 1.0)
        out_ref[:, h * hid : (h + 1) * hid] = o


def _gat_layer(x, adj_bf, W, a_src, a_dst):
    n = x.shape[0]
    fin = x.shape[1]
    heads, _, hid = W.shape
    bm = min(_BM, n)

    whe, g, eb, ed, wh_mean = pl.pallas_call(
        functools.partial(_proj_body, hid=hid),
        grid=(heads,),
        in_specs=[
            pl.BlockSpec((n, fin), lambda h: (0, 0)),
            pl.BlockSpec((1, fin, hid), lambda h: (h, 0, 0)),
            pl.BlockSpec((heads, hid), lambda h: (0, 0)),
            pl.BlockSpec((heads, hid), lambda h: (0, 0)),
        ],
        out_specs=[
            pl.BlockSpec((1, n, 2 * hid), lambda h: (h, 0, 0)),
            pl.BlockSpec((1, n, 1), lambda h: (h, 0, 0)),
            pl.BlockSpec((1, n, 1), lambda h: (h, 0, 0)),
            pl.BlockSpec((1, n, 1), lambda h: (h, 0, 0)),
            pl.BlockSpec((1, 1, hid), lambda h: (h, 0, 0)),
        ],
        out_shape=[
            jax.ShapeDtypeStruct((heads, n, 2 * hid), jnp.bfloat16),
            jax.ShapeDtypeStruct((heads, n, 1), jnp.float32),
            jax.ShapeDtypeStruct((heads, n, 1), jnp.float32),
            jax.ShapeDtypeStruct((heads, n, 1), jnp.float32),
            jax.ShapeDtypeStruct((heads, 1, hid), jnp.float32),
        ],
    )(x, W, a_src, a_dst)
    g = g.astype(jnp.bfloat16)
    eb = eb.reshape(heads, 1, n).astype(jnp.bfloat16)
    ed = ed.reshape(heads, 1, n).astype(jnp.bfloat16)

    out = pl.pallas_call(
        functools.partial(_attn_body, bm=bm, heads=heads, hid=hid),
        grid=(n // bm,),
        in_specs=[
            pl.BlockSpec((bm, n), lambda i: (i, 0)),
            pl.BlockSpec((heads, n, 2 * hid), lambda i: (0, 0, 0)),
            pl.BlockSpec((heads, n, 1), lambda i: (0, 0, 0)),
            pl.BlockSpec((heads, 1, n), lambda i: (0, 0, 0)),
            pl.BlockSpec((heads, 1, n), lambda i: (0, 0, 0)),
            pl.BlockSpec((heads, 1, hid), lambda i: (0, 0, 0)),
        ],
        out_specs=pl.BlockSpec((bm, heads * hid), lambda i: (i, 0)),
        out_shape=jax.ShapeDtypeStruct((n, heads * hid), jnp.float32),
    )(adj_bf, whe, g, eb, ed, wh_mean)
    return out


@jax.jit
def kernel(features, adj, W1, a_src1, a_dst1, W2, a_src2, a_dst2, W3, a_src3, a_dst3):
    adj_bf = adj.astype(jnp.bfloat16)
    x = _gat_layer(features, adj_bf, W1, a_src1, a_dst1)
    x = _gat_layer(x, adj_bf, W2, a_src2, a_dst2)
    x = _gat_layer(x, adj_bf, W3, a_src3, a_dst3)
    return x


# speedup vs baseline: 1.0247x; 1.0037x over previous
import functools

import jax
import jax.numpy as jnp
from jax.experimental import pallas as pl

_BM = 512


def _proj_body(x_ref, w_ref, asrc_ref, adst_ref,
               whe_ref, g_ref, eb_ref, ed_ref, mean_ref, *, hid):
    n = x_ref.shape[0]
    h = pl.program_id(0)
    wh = jnp.dot(x_ref[...], w_ref[0], preferred_element_type=jnp.float32)
    whe_ref[0, :, :hid] = wh.astype(jnp.bfloat16)
    col = jax.lax.broadcasted_iota(jnp.int32, (n, hid), 1)
    whe_ref[0, :, hid:] = jnp.where(col == 0, 1.0, 0.0).astype(jnp.bfloat16)
    asrc = asrc_ref[h][:, None]
    adst = adst_ref[h][:, None]
    aboth = jnp.concatenate([asrc, adst], axis=1)
    fs = jnp.dot(wh, aboth, preferred_element_type=jnp.float32)
    f_src = fs[:, 0:1]
    f_dst = fs[:, 1:2]
    g_ref[0] = jnp.exp(-0.8 * f_src)
    eb_ref[0] = jnp.exp(f_dst)
    ed_ref[0] = jnp.exp(0.2 * f_dst)
    mean_ref[0] = jnp.mean(wh, axis=0, keepdims=True)


def _attn_body(adj_ref, whe_ref, g_ref, eb_ref, ed_ref, mean_ref,
               out_ref, *, bm, heads, hid):
    i = pl.program_id(0)
    adj = adj_ref[...]
    for h in range(heads):
        g = g_ref[h, pl.ds(i * bm, bm), :]
        eb = eb_ref[h]
        ed = ed_ref[h]
        q = jnp.maximum(eb, g * ed)
        p = adj * q
        pv = jnp.dot(p, whe_ref[h], preferred_element_type=jnp.float32)
        denom = pv[:, hid : hid + 1]
        o = jnp.where(denom > 0.0, pv[:, :hid] / denom, mean_ref[h])
        o = jnp.where(o > 0.0, o, jnp.exp(o) - 1.0)
        out_ref[:, h * hid : (h + 1) * hid] = o


def _gat_layer(x, adj_bf, W, a_src, a_dst):
    n = x.shape[0]
    fin = x.shape[1]
    heads, _, hid = W.shape
    bm = min(_BM, n)

    whe, g, eb, ed, wh_mean = pl.pallas_call(
        functools.partial(_proj_body, hid=hid),
        grid=(heads,),
        in_specs=[
            pl.BlockSpec((n, fin), lambda h: (0, 0)),
            pl.BlockSpec((1, fin, hid), lambda h: (h, 0, 0)),
            pl.BlockSpec((heads, hid), lambda h: (0, 0)),
            pl.BlockSpec((heads, hid), lambda h: (0, 0)),
        ],
        out_specs=[
            pl.BlockSpec((1, n, 2 * hid), lambda h: (h, 0, 0)),
            pl.BlockSpec((1, n, 1), lambda h: (h, 0, 0)),
            pl.BlockSpec((1, n, 1), lambda h: (h, 0, 0)),
            pl.BlockSpec((1, n, 1), lambda h: (h, 0, 0)),
            pl.BlockSpec((1, 1, hid), lambda h: (h, 0, 0)),
        ],
        out_shape=[
            jax.ShapeDtypeStruct((heads, n, 2 * hid), jnp.bfloat16),
            jax.ShapeDtypeStruct((heads, n, 1), jnp.float32),
            jax.ShapeDtypeStruct((heads, n, 1), jnp.float32),
            jax.ShapeDtypeStruct((heads, n, 1), jnp.float32),
            jax.ShapeDtypeStruct((heads, 1, hid), jnp.float32),
        ],
    )(x, W, a_src, a_dst)
    g = g.astype(jnp.bfloat16)
    eb = eb.reshape(heads, 1, n).astype(jnp.bfloat16)
    ed = ed.reshape(heads, 1, n).astype(jnp.bfloat16)

    out = pl.pallas_call(
        functools.partial(_attn_body, bm=bm, heads=heads, hid=hid),
        grid=(n // bm,),
        in_specs=[
            pl.BlockSpec((bm, n), lambda i: (i, 0)),
            pl.BlockSpec((heads, n, 2 * hid), lambda i: (0, 0, 0)),
            pl.BlockSpec((heads, n, 1), lambda i: (0, 0, 0)),
            pl.BlockSpec((heads, 1, n), lambda i: (0, 0, 0)),
            pl.BlockSpec((heads, 1, n), lambda i: (0, 0, 0)),
            pl.BlockSpec((heads, 1, hid), lambda i: (0, 0, 0)),
        ],
        out_specs=pl.BlockSpec((bm, heads * hid), lambda i: (i, 0)),
        out_shape=jax.ShapeDtypeStruct((n, heads * hid), jnp.float32),
    )(adj_bf, whe, g, eb, ed, wh_mean)
    return out


@jax.jit
def kernel(features, adj, W1, a_src1, a_dst1, W2, a_src2, a_dst2, W3, a_src3, a_dst3):
    adj_bf = adj.astype(jnp.bfloat16)
    x = _gat_layer(features, adj_bf, W1, a_src1, a_dst1)
    x = _gat_layer(x, adj_bf, W2, a_src2, a_dst2)
    x = _gat_layer(x, adj_bf, W3, a_src3, a_dst3)
    return x
